# baseline (speedup 1.0000x reference)
import functools

import jax
import jax.numpy as jnp
from jax import lax
from jax.experimental import pallas as pl
from jax.experimental.pallas import tpu as pltpu
from jax.experimental.pallas import tpu_sc as plsc

N = 10000
E = 320000
D = 128
H = 128
C = 7

NC = 2
NS = 16
DG = D // NC
NPAD = 10240
K = 128
EPAD = 327680
EP = EPAD // NS
NPAIR = EP // (2 * K)
RPT = NPAD // NS

NROWBUF = 4
NMETA = 4


def _make_sage_sc(with_counts):
    mesh = plsc.VectorSubcoreMesh(core_axis_name="c", subcore_axis_name="s")

    @functools.partial(
        pl.kernel,
        mesh=mesh,
        compiler_params=pltpu.CompilerParams(
            use_tc_tiling_on_sc=False, needs_layout_passes=False),
        out_type=(
            [jax.ShapeDtypeStruct((NC, NPAD, DG), jnp.bfloat16)]
            + ([jax.ShapeDtypeStruct((NC, NPAD), jnp.float32)]
               if with_counts else [])
        ),
        scratch_types=(
            [
                pltpu.VMEM((K,), jnp.float32),
                pltpu.VMEM((K,), jnp.float32),
            ]
            + [pltpu.VMEM((6, K), jnp.int32)] * NMETA
            + [pltpu.VMEM((K, DG), jnp.bfloat16)] * NROWBUF
            + [
                pltpu.VMEM_SHARED((NPAD, DG), jnp.bfloat16),
                pltpu.VMEM_SHARED((NPAD, DG), jnp.bfloat16),
                pltpu.VMEM_SHARED((NPAD,), jnp.float32),
            ]
            + [pltpu.SemaphoreType.DMA] * (NMETA + 2 * NROWBUF)
        ),
    )
    def sage_aggregate(x_hbm, meta_hbm, *out_and_rest):
        if with_counts:
            out_hbm, cnt_out_hbm, z_v, one_v = out_and_rest[:4]
        else:
            out_hbm = out_and_rest[0]
            cnt_out_hbm = None
            z_v, one_v = out_and_rest[1:3]
        rest = out_and_rest[4:] if with_counts else out_and_rest[3:]
        meta = rest[:NMETA]
        rows = rest[NMETA:NMETA + NROWBUF]
        acc_sh = rest[NMETA + NROWBUF]
        x_sh = rest[NMETA + NROWBUF + 1]
        cnt_sh = rest[NMETA + NROWBUF + 2]
        sems = rest[NMETA + NROWBUF + 3:]
        sm = sems[:NMETA]
        sg = sems[NMETA:NMETA + NROWBUF]
        ss = sems[NMETA + NROWBUF:]

        cid = lax.axis_index("c")
        sid = lax.axis_index("s")
        cbase = sid * NPAIR
        roff = cid * NPAD

        zero32 = jnp.zeros((32,), jnp.bfloat16)
        zero16 = jnp.zeros((16,), jnp.float32)
        one16 = jnp.ones((16,), jnp.float32)

        def zrow(i, carry):
            for f in range(DG // 32):
                rows[0][i, pl.ds(f * 32, 32)] = zero32
            return carry
        lax.fori_loop(0, K, zrow, 0)
        for f in range(K // 16):
            z_v[pl.ds(f * 16, 16)] = zero16
            one_v[pl.ds(f * 16, 16)] = one16
        rbase = sid * RPT
        pltpu.async_copy(x_hbm.at[pl.ds(roff + rbase, RPT)],
                         x_sh.at[pl.ds(rbase, RPT)], sg[1])
        for j in range(RPT // K):
            pltpu.async_copy(rows[0], acc_sh.at[pl.ds(rbase + j * K, K)],
                             sg[0])
            if with_counts:
                pltpu.async_copy(z_v, cnt_sh.at[pl.ds(rbase + j * K, K)],
                                 ss[0])
        for j in range(RPT // K):
            pltpu.make_async_copy(rows[0], acc_sh.at[pl.ds(rbase, K)],
                                  sg[0]).wait()
            if with_counts:
                pltpu.make_async_copy(z_v, cnt_sh.at[pl.ds(rbase, K)],
                                      ss[0]).wait()
        pltpu.make_async_copy(x_hbm.at[pl.ds(roff + rbase, RPT)],
                              x_sh.at[pl.ds(rbase, RPT)], sg[1]).wait()
        plsc.subcore_barrier()

        def meta_fetch(c, q):
            pltpu.async_copy(meta_hbm.at[cbase + c], meta[q], sm[q])

        def meta_wait(q):
            pltpu.make_async_copy(meta_hbm.at[cbase], meta[q], sm[q]).wait()

        meta_fetch(0, 0)
        meta_fetch(1, 1)
        meta_wait(0)
        pltpu.async_copy(x_sh.at[meta[0].at[0]], rows[0], sg[0])
        pltpu.async_copy(x_sh.at[meta[0].at[1]], rows[1], sg[1])

        def drain_pair(bufs, q_any):
            for j, bb in enumerate(bufs):
                pltpu.make_async_copy(rows[bb],
                                      acc_sh.at[meta[q_any].at[2 + j]],
                                      ss[bb]).wait()
                if with_counts:
                    pltpu.make_async_copy(one_v,
                                          cnt_sh.at[meta[q_any].at[2 + j]],
                                          ss[bb]).wait()

        def mul_buf(bufi, q, wrow):
            wrow16 = jnp.full((16,), wrow, jnp.int32)

            def mul_row(e, carry2):
                wi = plsc.load_gather(
                    meta[q], [wrow16, jnp.full((16,), e, jnp.int32)])
                w = plsc.bitcast(wi, jnp.float32)
                wb = plsc.pack(w, w, format=plsc.PackFormat.INTERLEAVED)
                for f in range(DG // 32):
                    sl = pl.ds(f * 32, 32)
                    rows[bufi][e, sl] = rows[bufi][e, sl] * wb
                return carry2
            lax.fori_loop(0, K, mul_row, 0, unroll=8)

        def pair_step(pidx, b0, q, qn):
            b1 = b0 + 1
            n0 = 2 - b0
            n1 = n0 + 1

            @pl.when(pidx + 2 < NPAIR)
            def _():
                meta_fetch(pidx + 2, (q + 2) % NMETA)

            @pl.when(pidx >= 1)
            def _():
                drain_pair((n0, n1), 0)

            @pl.when(pidx + 1 < NPAIR)
            def _():
                meta_wait(qn)
                pltpu.async_copy(x_sh.at[meta[qn].at[0]], rows[n0], sg[n0])
                pltpu.async_copy(x_sh.at[meta[qn].at[1]], rows[n1], sg[n1])

            pltpu.make_async_copy(x_sh.at[meta[0].at[0]], rows[b0],
                                  sg[b0]).wait()
            pltpu.make_async_copy(x_sh.at[meta[0].at[1]], rows[b1],
                                  sg[b1]).wait()
            mul_buf(b0, q, 4)
            mul_buf(b1, q, 5)

            pltpu.async_copy(rows[b0], acc_sh.at[meta[q].at[2]], ss[b0],
                             add=True)
            pltpu.async_copy(rows[b1], acc_sh.at[meta[q].at[3]], ss[b1],
                             add=True)
            if with_counts:
                pltpu.async_copy(one_v, cnt_sh.at[meta[q].at[2]], ss[b0],
                                 add=True)
                pltpu.async_copy(one_v, cnt_sh.at[meta[q].at[3]], ss[b1],
                                 add=True)

        def round_body(r, carry):
            g = r * NMETA
            for j in range(NMETA):
                pair_step(g + j, 2 * (j % 2), j, (j + 1) % NMETA)
            return carry
        lax.fori_loop(0, NPAIR // NMETA, round_body, 0)

        bl_ = 2 * ((NPAIR - 1) % 2)
        drain_pair((bl_, bl_ + 1), 0)

        plsc.subcore_barrier()
        pltpu.async_copy(acc_sh.at[pl.ds(rbase, RPT)],
                         out_hbm.at[cid, pl.ds(rbase, RPT)], sg[0])
        if with_counts:
            pltpu.async_copy(cnt_sh.at[pl.ds(rbase, RPT)],
                             cnt_out_hbm.at[cid, pl.ds(rbase, RPT)], sg[1])
        pltpu.make_async_copy(acc_sh.at[pl.ds(rbase, RPT)],
                              out_hbm.at[cid, pl.ds(rbase, RPT)],
                              sg[0]).wait()
        if with_counts:
            pltpu.make_async_copy(cnt_sh.at[pl.ds(rbase, RPT)],
                                  cnt_out_hbm.at[cid, pl.ds(rbase, RPT)],
                                  sg[1]).wait()

    return sage_aggregate


_sage_sc_cnt = _make_sage_sc(True)
_sage_sc_nocnt = _make_sage_sc(False)


def _mean(slo_ref, shi_ref, c0_ref):
    inv = 1.0 / jnp.maximum(c0_ref[...], 1.0)
    mlo = slo_ref[0].astype(jnp.float32) * inv
    mhi = shi_ref[0].astype(jnp.float32) * inv
    return mlo, mhi


def _tc_layer_body(xlo_ref, xhi_ref, slo_ref, shi_ref, c0_ref,
                   wtl_ref, wth_ref, wbl_ref, wbh_ref, b_ref,
                   o_ref, obf_ref):
    mlo, mhi = _mean(slo_ref, shi_ref, c0_ref)
    dot = functools.partial(jnp.dot, preferred_element_type=jnp.float32)
    h = (dot(xlo_ref[0], wtl_ref[...])
         + dot(xhi_ref[0], wth_ref[...])
         + dot(mlo, wbl_ref[...])
         + dot(mhi, wbh_ref[...])
         + b_ref[...])
    h = jnp.maximum(h, 0.0)
    o_ref[0] = h[:, :DG]
    o_ref[1] = h[:, DG:]
    obf_ref[0] = h[:, :DG].astype(jnp.bfloat16)
    obf_ref[1] = h[:, DG:].astype(jnp.bfloat16)


def _tc_layer(x3, slo, shi, c0, W, b):
    B = 1024
    b2 = b.reshape(1, H)
    c0 = c0.reshape(NPAD, 1)
    grid = NPAD // B
    s3lo = slo.reshape(1, NPAD, DG)
    s3hi = shi.reshape(1, NPAD, DG)
    return pl.pallas_call(
        _tc_layer_body,
        grid=(grid,),
        in_specs=[
            pl.BlockSpec((1, B, DG), lambda i: (0, i, 0)),
            pl.BlockSpec((1, B, DG), lambda i: (1, i, 0)),
            pl.BlockSpec((1, B, DG), lambda i: (0, i, 0)),
            pl.BlockSpec((1, B, DG), lambda i: (0, i, 0)),
            pl.BlockSpec((B, 1), lambda i: (i, 0)),
            pl.BlockSpec((DG, H), lambda i: (0, 0)),
            pl.BlockSpec((DG, H), lambda i: (0, 0)),
            pl.BlockSpec((DG, H), lambda i: (0, 0)),
            pl.BlockSpec((DG, H), lambda i: (0, 0)),
            pl.BlockSpec((1, H), lambda i: (0, 0)),
        ],
        out_specs=[
            pl.BlockSpec((NC, B, DG), lambda i: (0, i, 0)),
            pl.BlockSpec((NC, B, DG), lambda i: (0, i, 0)),
        ],
        out_shape=[
            jax.ShapeDtypeStruct((NC, NPAD, DG), jnp.float32),
            jax.ShapeDtypeStruct((NC, NPAD, DG), jnp.bfloat16),
        ],
    )(x3, x3, s3lo, s3hi, c0, W[:DG], W[DG:D], W[D:D + DG], W[D + DG:], b2)


def _tc_layer3_body(x2lo_ref, x2hi_ref, slo_ref, shi_ref, c0_ref,
                    wtl_ref, wth_ref, wbl_ref, wbh_ref, b_ref,
                    x1lo_ref, x1hi_ref,
                    wl1l_ref, wl1h_ref, wl2l_ref, wl2h_ref, wl3_ref,
                    blp_ref, o_ref):
    mlo, mhi = _mean(slo_ref, shi_ref, c0_ref)
    dot = functools.partial(jnp.dot, preferred_element_type=jnp.float32)
    h3 = (dot(x2lo_ref[0], wtl_ref[...])
          + dot(x2hi_ref[0], wth_ref[...])
          + dot(mlo, wbl_ref[...])
          + dot(mhi, wbh_ref[...])
          + b_ref[...])
    h3 = jnp.maximum(h3, 0.0)
    z = (dot(x1lo_ref[0], wl1l_ref[...])
         + dot(x1hi_ref[0], wl1h_ref[...])
         + dot(x2lo_ref[0], wl2l_ref[...])
         + dot(x2hi_ref[0], wl2h_ref[...])
         + dot(h3, wl3_ref[...])
         + blp_ref[...])
    m = jnp.max(z, axis=-1, keepdims=True)
    ez = jnp.exp(z - m)
    sz = jnp.sum(ez, axis=-1, keepdims=True)
    o_ref[...] = z - m - jnp.log(sz)


def _tc_layer3(x1, x2, slo, shi, c0, W, b, Wl, bl):
    B = 1024
    CP = 128
    w_pad = jnp.zeros((3 * H, CP), jnp.float32).at[:, :C].set(Wl)
    b_pad = jnp.full((1, CP), -1e30, jnp.float32).at[0, :C].set(bl)
    b2 = b.reshape(1, H)
    c0 = c0.reshape(NPAD, 1)
    grid = NPAD // B
    s3lo = slo.reshape(1, NPAD, DG)
    s3hi = shi.reshape(1, NPAD, DG)
    lo = lambda i: (0, i, 0)
    hi = lambda i: (1, i, 0)
    const = lambda i: (0, 0)
    return pl.pallas_call(
        _tc_layer3_body,
        grid=(grid,),
        in_specs=[
            pl.BlockSpec((1, B, DG), lo),
            pl.BlockSpec((1, B, DG), hi),
            pl.BlockSpec((1, B, DG), lo),
            pl.BlockSpec((1, B, DG), lo),
            pl.BlockSpec((B, 1), lambda i: (i, 0)),
            pl.BlockSpec((DG, H), const),
            pl.BlockSpec((DG, H), const),
            pl.BlockSpec((DG, H), const),
            pl.BlockSpec((DG, H), const),
            pl.BlockSpec((1, H), const),
            pl.BlockSpec((1, B, DG), lo),
            pl.BlockSpec((1, B, DG), hi),
            pl.BlockSpec((DG, CP), const),
            pl.BlockSpec((DG, CP), const),
            pl.BlockSpec((DG, CP), const),
            pl.BlockSpec((DG, CP), const),
            pl.BlockSpec((H, CP), const),
            pl.BlockSpec((1, CP), const),
        ],
        out_specs=pl.BlockSpec((B, CP), lambda i: (i, 0)),
        out_shape=jax.ShapeDtypeStruct((NPAD, CP), jnp.float32),
    )(x2, x2, s3lo, s3hi, c0,
      W[:DG], W[DG:D], W[D:D + DG], W[D + DG:], b2,
      x1, x1,
      w_pad[:DG], w_pad[DG:D], w_pad[D:D + DG], w_pad[D + DG:2 * D],
      w_pad[2 * D:], b_pad)


def kernel(x, edge_index, edge_weight, W1, b1, W2, b2, W3, b3, Wl, bl):
    x = x.astype(jnp.float32)
    xp = jnp.zeros((NPAD, D), jnp.float32).at[:N].set(x)
    x0 = jnp.stack([xp[:, :DG], xp[:, DG:]])
    x0bf = x0.astype(jnp.bfloat16)
    pad = EPAD - E
    src = jnp.pad(edge_index[0].astype(jnp.int32), (0, pad))
    dst = jnp.pad(edge_index[1].astype(jnp.int32), (0, pad),
                  constant_values=NPAD - 1)
    wgt = jnp.pad(edge_weight.astype(jnp.float32), (0, pad))
    meta = jnp.concatenate(
        [src.reshape(EPAD // (2 * K), 2, K),
         dst.reshape(EPAD // (2 * K), 2, K),
         lax.bitcast_convert_type(wgt, jnp.int32).reshape(
             EPAD // (2 * K), 2, K)],
        axis=1)

    s1, c1 = _sage_sc_cnt(x0bf.reshape(NC * NPAD, DG), meta)
    x1, x1bf = _tc_layer(x0, s1[0], s1[1], c1[0], W1, b1)
    s2 = _sage_sc_nocnt(x1bf.reshape(NC * NPAD, DG), meta)[0]
    x2, x2bf = _tc_layer(x1, s2[0], s2[1], c1[0], W2, b2)
    s3 = _sage_sc_nocnt(x2bf.reshape(NC * NPAD, DG), meta)[0]
    out = _tc_layer3(x1, x2, s3[0], s3[1], c1[0], W3, b3, Wl, bl)
    return out[:N, :C]


# speedup vs baseline: 1.0491x; 1.0491x over previous
import functools

import jax
import jax.numpy as jnp
from jax import lax
from jax.experimental import pallas as pl
from jax.experimental.pallas import tpu as pltpu
from jax.experimental.pallas import tpu_sc as plsc

N = 10000
E = 320000
D = 128
H = 128
C = 7

NC = 2
NS = 16
DG = D // NC
NPAD = 10240
K = 128
EPAD = 327680
EP = EPAD // NS
NPAIR = EP // (2 * K)
RPT = NPAD // NS

NROWBUF = 4
NMETA = 4


def _make_sage_sc(with_counts):
    mesh = plsc.VectorSubcoreMesh(core_axis_name="c", subcore_axis_name="s")

    @functools.partial(
        pl.kernel,
        mesh=mesh,
        compiler_params=pltpu.CompilerParams(
            use_tc_tiling_on_sc=False, needs_layout_passes=False),
        out_type=(
            [jax.ShapeDtypeStruct((NC, NPAD, DG), jnp.bfloat16)]
            + ([jax.ShapeDtypeStruct((NC, NPAD), jnp.float32)]
               if with_counts else [])
        ),
        scratch_types=(
            [
                pltpu.VMEM((K,), jnp.float32),
                pltpu.VMEM((K,), jnp.float32),
            ]
            + [pltpu.VMEM((6, K), jnp.int32)] * NMETA
            + [pltpu.VMEM((K, DG), jnp.bfloat16)] * NROWBUF
            + [
                pltpu.VMEM_SHARED((NPAD, DG), jnp.bfloat16),
                pltpu.VMEM_SHARED((NPAD, DG), jnp.bfloat16),
                pltpu.VMEM_SHARED((NPAD,), jnp.float32),
            ]
            + [pltpu.SemaphoreType.DMA] * (NMETA + 2 * NROWBUF)
        ),
    )
    def sage_aggregate(x_hbm, meta_hbm, *out_and_rest):
        if with_counts:
            out_hbm, cnt_out_hbm, z_v, one_v = out_and_rest[:4]
        else:
            out_hbm = out_and_rest[0]
            cnt_out_hbm = None
            z_v, one_v = out_and_rest[1:3]
        rest = out_and_rest[4:] if with_counts else out_and_rest[3:]
        meta = rest[:NMETA]
        rows = rest[NMETA:NMETA + NROWBUF]
        acc_sh = rest[NMETA + NROWBUF]
        x_sh = rest[NMETA + NROWBUF + 1]
        cnt_sh = rest[NMETA + NROWBUF + 2]
        sems = rest[NMETA + NROWBUF + 3:]
        sm = sems[:NMETA]
        sg = sems[NMETA:NMETA + NROWBUF]
        ss = sems[NMETA + NROWBUF:]

        cid = lax.axis_index("c")
        sid = lax.axis_index("s")
        cbase = sid * NPAIR
        roff = cid * NPAD

        zero32 = jnp.zeros((32,), jnp.bfloat16)
        zero16 = jnp.zeros((16,), jnp.float32)
        one16 = jnp.ones((16,), jnp.float32)

        def zrow(i, carry):
            for f in range(DG // 32):
                rows[0][i, pl.ds(f * 32, 32)] = zero32
            return carry
        lax.fori_loop(0, K, zrow, 0)
        for f in range(K // 16):
            z_v[pl.ds(f * 16, 16)] = zero16
            one_v[pl.ds(f * 16, 16)] = one16
        rbase = sid * RPT
        pltpu.async_copy(x_hbm.at[pl.ds(roff + rbase, RPT)],
                         x_sh.at[pl.ds(rbase, RPT)], sg[1])
        for j in range(RPT // K):
            pltpu.async_copy(rows[0], acc_sh.at[pl.ds(rbase + j * K, K)],
                             sg[0])
            if with_counts:
                pltpu.async_copy(z_v, cnt_sh.at[pl.ds(rbase + j * K, K)],
                                 ss[0])
        for j in range(RPT // K):
            pltpu.make_async_copy(rows[0], acc_sh.at[pl.ds(rbase, K)],
                                  sg[0]).wait()
            if with_counts:
                pltpu.make_async_copy(z_v, cnt_sh.at[pl.ds(rbase, K)],
                                      ss[0]).wait()
        pltpu.make_async_copy(x_hbm.at[pl.ds(roff + rbase, RPT)],
                              x_sh.at[pl.ds(rbase, RPT)], sg[1]).wait()
        plsc.subcore_barrier()

        def meta_fetch(c, q):
            pltpu.async_copy(meta_hbm.at[cbase + c], meta[q], sm[q])

        def meta_wait(q):
            pltpu.make_async_copy(meta_hbm.at[cbase], meta[q], sm[q]).wait()

        meta_fetch(0, 0)
        meta_fetch(1, 1)
        meta_wait(0)
        pltpu.async_copy(x_sh.at[meta[0].at[0]], rows[0], sg[0])
        pltpu.async_copy(x_sh.at[meta[0].at[1]], rows[1], sg[1])

        def drain_pair(bufs, q_any):
            for j, bb in enumerate(bufs):
                pltpu.make_async_copy(rows[bb],
                                      acc_sh.at[meta[q_any].at[2 + j]],
                                      ss[bb]).wait()
                if with_counts:
                    pltpu.make_async_copy(one_v,
                                          cnt_sh.at[meta[q_any].at[2 + j]],
                                          ss[bb]).wait()

        def mul_buf(bufi, q, wrow):
            wrow16 = jnp.full((16,), wrow, jnp.int32)

            def mul_row(e, carry2):
                wi = plsc.load_gather(
                    meta[q], [wrow16, jnp.full((16,), e, jnp.int32)])
                w = plsc.bitcast(wi, jnp.float32)
                wb = plsc.pack(w, w, format=plsc.PackFormat.INTERLEAVED)
                for f in range(DG // 32):
                    sl = pl.ds(f * 32, 32)
                    rows[bufi][e, sl] = rows[bufi][e, sl] * wb
                return carry2
            lax.fori_loop(0, K, mul_row, 0, unroll=4)

        def pair_step(pidx, b0, q, qn):
            b1 = b0 + 1
            n0 = 2 - b0
            n1 = n0 + 1

            @pl.when(pidx + 2 < NPAIR)
            def _():
                meta_fetch(pidx + 2, (q + 2) % NMETA)

            @pl.when(pidx >= 1)
            def _():
                drain_pair((n0, n1), 0)

            @pl.when(pidx + 1 < NPAIR)
            def _():
                meta_wait(qn)
                pltpu.async_copy(x_sh.at[meta[qn].at[0]], rows[n0], sg[n0])
                pltpu.async_copy(x_sh.at[meta[qn].at[1]], rows[n1], sg[n1])

            pltpu.make_async_copy(x_sh.at[meta[0].at[0]], rows[b0],
                                  sg[b0]).wait()
            pltpu.make_async_copy(x_sh.at[meta[0].at[1]], rows[b1],
                                  sg[b1]).wait()
            mul_buf(b0, q, 4)
            mul_buf(b1, q, 5)

            pltpu.async_copy(rows[b0], acc_sh.at[meta[q].at[2]], ss[b0],
                             add=True)
            pltpu.async_copy(rows[b1], acc_sh.at[meta[q].at[3]], ss[b1],
                             add=True)
            if with_counts:
                pltpu.async_copy(one_v, cnt_sh.at[meta[q].at[2]], ss[b0],
                                 add=True)
                pltpu.async_copy(one_v, cnt_sh.at[meta[q].at[3]], ss[b1],
                                 add=True)

        def round_body(r, carry):
            g = r * NMETA
            for j in range(NMETA):
                pair_step(g + j, 2 * (j % 2), j, (j + 1) % NMETA)
            return carry
        lax.fori_loop(0, NPAIR // NMETA, round_body, 0)

        bl_ = 2 * ((NPAIR - 1) % 2)
        drain_pair((bl_, bl_ + 1), 0)

        plsc.subcore_barrier()
        pltpu.async_copy(acc_sh.at[pl.ds(rbase, RPT)],
                         out_hbm.at[cid, pl.ds(rbase, RPT)], sg[0])
        if with_counts:
            pltpu.async_copy(cnt_sh.at[pl.ds(rbase, RPT)],
                             cnt_out_hbm.at[cid, pl.ds(rbase, RPT)], sg[1])
        pltpu.make_async_copy(acc_sh.at[pl.ds(rbase, RPT)],
                              out_hbm.at[cid, pl.ds(rbase, RPT)],
                              sg[0]).wait()
        if with_counts:
            pltpu.make_async_copy(cnt_sh.at[pl.ds(rbase, RPT)],
                                  cnt_out_hbm.at[cid, pl.ds(rbase, RPT)],
                                  sg[1]).wait()

    return sage_aggregate


_sage_sc_cnt = _make_sage_sc(True)
_sage_sc_nocnt = _make_sage_sc(False)


def _mean(slo_ref, shi_ref, c0_ref):
    inv = 1.0 / jnp.maximum(c0_ref[...], 1.0)
    mlo = slo_ref[0].astype(jnp.float32) * inv
    mhi = shi_ref[0].astype(jnp.float32) * inv
    return mlo, mhi


def _tc_layer_body(xlo_ref, xhi_ref, slo_ref, shi_ref, c0_ref,
                   wtl_ref, wth_ref, wbl_ref, wbh_ref, b_ref,
                   o_ref, obf_ref):
    mlo, mhi = _mean(slo_ref, shi_ref, c0_ref)
    dot = functools.partial(jnp.dot, preferred_element_type=jnp.float32)
    h = (dot(xlo_ref[0], wtl_ref[...])
         + dot(xhi_ref[0], wth_ref[...])
         + dot(mlo, wbl_ref[...])
         + dot(mhi, wbh_ref[...])
         + b_ref[...])
    h = jnp.maximum(h, 0.0)
    o_ref[0] = h[:, :DG]
    o_ref[1] = h[:, DG:]
    obf_ref[0] = h[:, :DG].astype(jnp.bfloat16)
    obf_ref[1] = h[:, DG:].astype(jnp.bfloat16)


def _tc_layer(x3, slo, shi, c0, W, b):
    B = 1024
    b2 = b.reshape(1, H)
    c0 = c0.reshape(NPAD, 1)
    grid = NPAD // B
    s3lo = slo.reshape(1, NPAD, DG)
    s3hi = shi.reshape(1, NPAD, DG)
    return pl.pallas_call(
        _tc_layer_body,
        grid=(grid,),
        in_specs=[
            pl.BlockSpec((1, B, DG), lambda i: (0, i, 0)),
            pl.BlockSpec((1, B, DG), lambda i: (1, i, 0)),
            pl.BlockSpec((1, B, DG), lambda i: (0, i, 0)),
            pl.BlockSpec((1, B, DG), lambda i: (0, i, 0)),
            pl.BlockSpec((B, 1), lambda i: (i, 0)),
            pl.BlockSpec((DG, H), lambda i: (0, 0)),
            pl.BlockSpec((DG, H), lambda i: (0, 0)),
            pl.BlockSpec((DG, H), lambda i: (0, 0)),
            pl.BlockSpec((DG, H), lambda i: (0, 0)),
            pl.BlockSpec((1, H), lambda i: (0, 0)),
        ],
        out_specs=[
            pl.BlockSpec((NC, B, DG), lambda i: (0, i, 0)),
            pl.BlockSpec((NC, B, DG), lambda i: (0, i, 0)),
        ],
        out_shape=[
            jax.ShapeDtypeStruct((NC, NPAD, DG), jnp.float32),
            jax.ShapeDtypeStruct((NC, NPAD, DG), jnp.bfloat16),
        ],
    )(x3, x3, s3lo, s3hi, c0, W[:DG], W[DG:D], W[D:D + DG], W[D + DG:], b2)


def _tc_layer3_body(x2lo_ref, x2hi_ref, slo_ref, shi_ref, c0_ref,
                    wtl_ref, wth_ref, wbl_ref, wbh_ref, b_ref,
                    x1lo_ref, x1hi_ref,
                    wl1l_ref, wl1h_ref, wl2l_ref, wl2h_ref, wl3_ref,
                    blp_ref, o_ref):
    mlo, mhi = _mean(slo_ref, shi_ref, c0_ref)
    dot = functools.partial(jnp.dot, preferred_element_type=jnp.float32)
    h3 = (dot(x2lo_ref[0], wtl_ref[...])
          + dot(x2hi_ref[0], wth_ref[...])
          + dot(mlo, wbl_ref[...])
          + dot(mhi, wbh_ref[...])
          + b_ref[...])
    h3 = jnp.maximum(h3, 0.0)
    z = (dot(x1lo_ref[0], wl1l_ref[...])
         + dot(x1hi_ref[0], wl1h_ref[...])
         + dot(x2lo_ref[0], wl2l_ref[...])
         + dot(x2hi_ref[0], wl2h_ref[...])
         + dot(h3, wl3_ref[...])
         + blp_ref[...])
    m = jnp.max(z, axis=-1, keepdims=True)
    ez = jnp.exp(z - m)
    sz = jnp.sum(ez, axis=-1, keepdims=True)
    o_ref[...] = z - m - jnp.log(sz)


def _tc_layer3(x1, x2, slo, shi, c0, W, b, Wl, bl):
    B = 1024
    CP = 128
    w_pad = jnp.zeros((3 * H, CP), jnp.float32).at[:, :C].set(Wl)
    b_pad = jnp.full((1, CP), -1e30, jnp.float32).at[0, :C].set(bl)
    b2 = b.reshape(1, H)
    c0 = c0.reshape(NPAD, 1)
    grid = NPAD // B
    s3lo = slo.reshape(1, NPAD, DG)
    s3hi = shi.reshape(1, NPAD, DG)
    lo = lambda i: (0, i, 0)
    hi = lambda i: (1, i, 0)
    const = lambda i: (0, 0)
    return pl.pallas_call(
        _tc_layer3_body,
        grid=(grid,),
        in_specs=[
            pl.BlockSpec((1, B, DG), lo),
            pl.BlockSpec((1, B, DG), hi),
            pl.BlockSpec((1, B, DG), lo),
            pl.BlockSpec((1, B, DG), lo),
            pl.BlockSpec((B, 1), lambda i: (i, 0)),
            pl.BlockSpec((DG, H), const),
            pl.BlockSpec((DG, H), const),
            pl.BlockSpec((DG, H), const),
            pl.BlockSpec((DG, H), const),
            pl.BlockSpec((1, H), const),
            pl.BlockSpec((1, B, DG), lo),
            pl.BlockSpec((1, B, DG), hi),
            pl.BlockSpec((DG, CP), const),
            pl.BlockSpec((DG, CP), const),
            pl.BlockSpec((DG, CP), const),
            pl.BlockSpec((DG, CP), const),
            pl.BlockSpec((H, CP), const),
            pl.BlockSpec((1, CP), const),
        ],
        out_specs=pl.BlockSpec((B, CP), lambda i: (i, 0)),
        out_shape=jax.ShapeDtypeStruct((NPAD, CP), jnp.float32),
    )(x2, x2, s3lo, s3hi, c0,
      W[:DG], W[DG:D], W[D:D + DG], W[D + DG:], b2,
      x1, x1,
      w_pad[:DG], w_pad[DG:D], w_pad[D:D + DG], w_pad[D + DG:2 * D],
      w_pad[2 * D:], b_pad)


def kernel(x, edge_index, edge_weight, W1, b1, W2, b2, W3, b3, Wl, bl):
    x = x.astype(jnp.float32)
    xp = jnp.zeros((NPAD, D), jnp.float32).at[:N].set(x)
    x0 = jnp.stack([xp[:, :DG], xp[:, DG:]])
    x0bf = x0.astype(jnp.bfloat16)
    pad = EPAD - E
    src = jnp.pad(edge_index[0].astype(jnp.int32), (0, pad))
    dst = jnp.pad(edge_index[1].astype(jnp.int32), (0, pad),
                  constant_values=NPAD - 1)
    wgt = jnp.pad(edge_weight.astype(jnp.float32), (0, pad))
    meta = jnp.concatenate(
        [src.reshape(EPAD // (2 * K), 2, K),
         dst.reshape(EPAD // (2 * K), 2, K),
         lax.bitcast_convert_type(wgt, jnp.int32).reshape(
             EPAD // (2 * K), 2, K)],
        axis=1)

    s1, c1 = _sage_sc_cnt(x0bf.reshape(NC * NPAD, DG), meta)
    x1, x1bf = _tc_layer(x0, s1[0], s1[1], c1[0], W1, b1)
    s2 = _sage_sc_nocnt(x1bf.reshape(NC * NPAD, DG), meta)[0]
    x2, x2bf = _tc_layer(x1, s2[0], s2[1], c1[0], W2, b2)
    s3 = _sage_sc_nocnt(x2bf.reshape(NC * NPAD, DG), meta)[0]
    out = _tc_layer3(x1, x2, s3[0], s3[1], c1[0], W3, b3, Wl, bl)
    return out[:N, :C]
